# baseline (speedup 1.0000x reference)
import jax
import jax.numpy as jnp
from jax import lax
from jax.experimental import pallas as pl
from jax.experimental.pallas import tpu as pltpu
from jax.experimental.pallas import tpu_sc as plsc

N = 10000
E = 160000
D = 256
DE = 16

_NC = 2
_NS = 16
_NW = _NC * _NS
_L = 16

_WIN = 128
_NWIN = E // _WIN
_NPAD = 10240
_ROWS_T = _NPAD // _NS

_NODE_BLK = 1000
_EDGE_BLK = 6400


def _node_proj_body(x_ref, w1, b1, w2, b2, w3, b3, w4, b4,
                    o1, o2, o3, o4):
    x = x_ref[...]
    dn = (((1,), (1,)), ((), ()))
    o1[...] = lax.dot_general(x, w1[...], dn,
                              preferred_element_type=jnp.float32) + b1[...]
    o2[...] = lax.dot_general(x, w2[...], dn,
                              preferred_element_type=jnp.float32) + b2[...]
    o3[...] = lax.dot_general(x, w3[...], dn,
                              preferred_element_type=jnp.float32) + b3[...]
    o4[...] = lax.dot_general(x, w4[...], dn,
                              preferred_element_type=jnp.float32) + b4[...]


def _node_projections(x, W1, b1, W2, b2, W3, b3, W4, b4):
    grid = (N // _NODE_BLK,)
    xspec = pl.BlockSpec((_NODE_BLK, D), lambda i: (i, 0))
    wspec = pl.BlockSpec((D, D), lambda i: (0, 0))
    bspec = pl.BlockSpec((1, D), lambda i: (0, 0))
    ospec = pl.BlockSpec((_NODE_BLK, D), lambda i: (i, 0))
    out_shape = [jax.ShapeDtypeStruct((N, D), jnp.float32)] * 4
    return pl.pallas_call(
        _node_proj_body,
        grid=grid,
        in_specs=[xspec, wspec, bspec, wspec, bspec, wspec, bspec, wspec,
                  bspec],
        out_specs=[ospec] * 4,
        out_shape=out_shape,
    )(x, W1, b1.reshape(1, D), W2, b2.reshape(1, D),
      W3, b3.reshape(1, D), W4, b4.reshape(1, D))


def _edge_score_body(u_ref, ef_ref, w_ref, b_ref, wd_ref, o_ref):
    dn = (((1,), (1,)), ((), ()))
    g = lax.dot_general(ef_ref[...], w_ref[...], dn,
                        preferred_element_type=jnp.float32) + b_ref[...]
    atn = jnp.maximum(u_ref[...] + g, 0.0)
    o_ref[...] = lax.dot_general(atn, wd_ref[...], (((1,), (0,)), ((), ())),
                                 preferred_element_type=jnp.float32)


def _edge_scores(u, ef, W_ae, b_ae, wdot):
    grid = (E // _EDGE_BLK,)
    return pl.pallas_call(
        _edge_score_body,
        grid=grid,
        in_specs=[pl.BlockSpec((_EDGE_BLK, D), lambda i: (i, 0)),
                  pl.BlockSpec((_EDGE_BLK, DE), lambda i: (i, 0)),
                  pl.BlockSpec((D, DE), lambda i: (0, 0)),
                  pl.BlockSpec((1, D), lambda i: (0, 0)),
                  pl.BlockSpec((D, 1), lambda i: (0, 0))],
        out_specs=pl.BlockSpec((_EDGE_BLK, 1), lambda i: (i, 0)),
        out_shape=jax.ShapeDtypeStruct((E, 1), jnp.float32),
    )(u, ef, W_ae, b_ae.reshape(1, D), wdot)


def _exr_body(ex_ref, o_ref):
    o_ref[...] = ex_ref[...] * jnp.ones((1, DE), jnp.float32)


def _ex_replicate(ex):
    blk = 8000
    return pl.pallas_call(
        _exr_body,
        grid=(E // blk,),
        in_specs=[pl.BlockSpec((blk, 1), lambda i: (i, 0))],
        out_specs=pl.BlockSpec((blk, DE), lambda i: (i, 0)),
        out_shape=jax.ShapeDtypeStruct((E, DE), jnp.float32),
    )(ex)


def _final_body(a0_ref, a1_ref, wgt_ref, t_ref, den_ref, w_ref, b_ref,
                o_ref):
    dn = (((1,), (1,)), ((), ()))
    tt = lax.dot_general(t_ref[...], w_ref[...], dn,
                         preferred_element_type=jnp.float32)
    agg = jnp.concatenate([a0_ref[...], a1_ref[...]], axis=1)
    den = den_ref[...]
    r = 1.0 / (den + 1e-12)
    o_ref[...] = wgt_ref[...] + (agg + tt + den * b_ref[...]) * r


def _final_combine(agg0, agg1, wgt, t16, denom, W_me, b_me):
    grid = (N // _NODE_BLK,)
    nspec = pl.BlockSpec((_NODE_BLK, D), lambda i: (i, 0))
    hspec = pl.BlockSpec((_NODE_BLK, _H), lambda i: (i, 0))
    return pl.pallas_call(
        _final_body,
        grid=grid,
        in_specs=[hspec, hspec, nspec,
                  pl.BlockSpec((_NODE_BLK, DE), lambda i: (i, 0)),
                  pl.BlockSpec((_NODE_BLK, 1), lambda i: (i, 0)),
                  pl.BlockSpec((D, DE), lambda i: (0, 0)),
                  pl.BlockSpec((1, D), lambda i: (0, 0))],
        out_specs=nspec,
        out_shape=jax.ShapeDtypeStruct((N, D), jnp.float32),
    )(agg0, agg1, wgt, t16, denom, W_me, b_me.reshape(1, D))


_SC_MESH = dict(core_axis_name="c", subcore_axis_name="s")


def _gather_add_body(asrc_hbm, adst_hbm, sidx_hbm, didx_hbm, u_hbm,
                     iv1, iv2, ra, rb, uv, sem1, sem2):
    wid = lax.axis_index("s") * _NC + lax.axis_index("c")
    nwin = (_NWIN - wid - 1) // _NW + 1

    def window(k, carry):
        base = (k * _NW + wid) * _WIN
        pltpu.sync_copy(sidx_hbm.at[pl.ds(base, _WIN)], iv1)
        cp1 = pltpu.async_copy(asrc_hbm.at[iv1], ra, sem1)
        pltpu.sync_copy(didx_hbm.at[pl.ds(base, _WIN)], iv2)
        cp2 = pltpu.async_copy(adst_hbm.at[iv2], rb, sem2)
        cp1.wait()
        cp2.wait()

        def edge(e, c):
            for j in range(D // _L):
                uv[e, pl.ds(j * _L, _L)] = (ra[e, pl.ds(j * _L, _L)] +
                                            rb[e, pl.ds(j * _L, _L)])
            return c

        lax.fori_loop(0, _WIN, edge, 0)
        pltpu.sync_copy(uv, u_hbm.at[pl.ds(base, _WIN)])
        return carry

    lax.fori_loop(0, nwin, window, 0)


def _sc_gather_add(a_src, a_dst, src, dst):
    mesh = plsc.VectorSubcoreMesh(**_SC_MESH)
    f = pl.kernel(
        _gather_add_body,
        out_type=jax.ShapeDtypeStruct((E, D), jnp.float32),
        mesh=mesh,
        scratch_types=[
            pltpu.VMEM((_WIN,), jnp.int32),
            pltpu.VMEM((_WIN,), jnp.int32),
            pltpu.VMEM((_WIN, D), jnp.float32),
            pltpu.VMEM((_WIN, D), jnp.float32),
            pltpu.VMEM((_WIN, D), jnp.float32),
            pltpu.SemaphoreType.DMA,
            pltpu.SemaphoreType.DMA,
        ],
    )
    return f(a_src, a_dst, src, dst)


def _softmax_denom_body(s_hbm, m_hbm, didx_hbm, ex_hbm, dp_hbm,
                        mv, sv, ev, iv, zv, shared, sem):
    cid = lax.axis_index("c")
    sid = lax.axis_index("s")
    wid = sid * _NC + cid
    nwin = (_NWIN - wid - 1) // _NW + 1

    def zrow(r, c):
        zv[pl.ds(r * _L, _L)] = jnp.zeros((_L,), jnp.float32)
        return c

    lax.fori_loop(0, _ROWS_T // _L, zrow, 0)
    pltpu.sync_copy(zv, shared.at[pl.ds(sid * _ROWS_T, _ROWS_T)])
    plsc.subcore_barrier()

    pltpu.sync_copy(m_hbm, mv)
    m = mv[...]

    def window(k, carry):
        base = (k * _NW + wid) * _WIN
        pltpu.sync_copy(s_hbm.at[pl.ds(base, _WIN)], sv)
        for q in range(_WIN // _L):
            ev[pl.ds(q * _L, _L)] = jnp.exp(sv[pl.ds(q * _L, _L)] - m)
        pltpu.sync_copy(ev, ex_hbm.at[pl.ds(base, _WIN)])
        pltpu.sync_copy(didx_hbm.at[pl.ds(base, _WIN)], iv)
        pltpu.sync_copy(ev, shared.at[iv], add=True)
        return carry

    lax.fori_loop(0, nwin, window, 0)
    plsc.subcore_barrier()
    pltpu.sync_copy(shared.at[pl.ds(sid * _ROWS_T, _ROWS_T)],
                    dp_hbm.at[cid, pl.ds(sid * _ROWS_T, _ROWS_T)])


def _sc_softmax_denom(scores, m16, dst):
    mesh = plsc.VectorSubcoreMesh(**_SC_MESH)
    f = pl.kernel(
        _softmax_denom_body,
        out_type=[jax.ShapeDtypeStruct((E,), jnp.float32),
                  jax.ShapeDtypeStruct((_NC, _NPAD), jnp.float32)],
        mesh=mesh,
        scratch_types=[
            pltpu.VMEM((_L,), jnp.float32),
            pltpu.VMEM((_WIN,), jnp.float32),
            pltpu.VMEM((_WIN,), jnp.float32),
            pltpu.VMEM((_WIN,), jnp.int32),
            pltpu.VMEM((_ROWS_T,), jnp.float32),
            pltpu.VMEM_SHARED((_NPAD,), jnp.float32),
            pltpu.SemaphoreType.DMA,
        ],
    )
    return f(scores, m16, dst)


_H = D // _NC


_NPAD_A = 10112
_RT_A = _NPAD_A // _NS


def _agg_body_impl(exr_hbm, m2_hbm, sidx_hbm, didx_hbm, agg_hbm,
                   iv1, iv2, exv, rv, zv, shared, sem):
    cid = lax.axis_index("c")
    sid = lax.axis_index("s")
    nwin = (_NWIN - sid - 1) // _NS + 1

    def zrow(r, c):
        for j in range(_H // _L):
            zv[r, pl.ds(j * _L, _L)] = jnp.zeros((_L,), jnp.float32)
        return c

    lax.fori_loop(0, _WIN, zrow, 0)

    def zcopy(r, c):
        pltpu.sync_copy(zv, shared.at[pl.ds(sid * _RT_A + r * _WIN, _WIN)])
        return c

    lax.fori_loop(0, _RT_A // _WIN, zcopy, 0)
    pltpu.sync_copy(zv.at[pl.ds(0, _RT_A % _WIN)],
                    shared.at[pl.ds(sid * _RT_A + (_RT_A // _WIN) * _WIN,
                                    _RT_A % _WIN)])
    plsc.subcore_barrier()

    def window(k, carry):
        base = (k * _NS + sid) * _WIN
        pltpu.sync_copy(sidx_hbm.at[pl.ds(base, _WIN)], iv1)

        def ixf(c, carry2):
            v = iv1[c * _L, pl.ds(0, _L)] if False else iv1[pl.ds(c * _L, _L)]
            iv1[pl.ds(c * _L, _L)] = v * 2 + cid
            return carry2

        lax.fori_loop(0, _WIN // _L, ixf, 0)
        cp = pltpu.async_copy(m2_hbm.at[iv1], rv, sem)
        pltpu.sync_copy(exr_hbm.at[pl.ds(base, _WIN)], exv)
        pltpu.sync_copy(didx_hbm.at[pl.ds(base, _WIN)], iv2)
        cp.wait()

        def edge(e, c):
            er = exv[e, pl.ds(0, _L)]
            for j in range(_H // _L):
                rv[e, pl.ds(j * _L, _L)] = rv[e, pl.ds(j * _L, _L)] * er
            return c

        lax.fori_loop(0, _WIN, edge, 0)
        pltpu.sync_copy(rv, shared.at[iv2], add=True)
        return carry

    lax.fori_loop(0, nwin, window, 0)
    plsc.subcore_barrier()
    pltpu.sync_copy(shared.at[pl.ds(sid * _RT_A, _RT_A)],
                    agg_hbm.at[cid, pl.ds(sid * _RT_A, _RT_A)])


def _sc_agg(exr, m2, src, dst):
    mesh = plsc.VectorSubcoreMesh(**_SC_MESH)
    f = pl.kernel(
        _agg_body_impl,
        out_type=jax.ShapeDtypeStruct((_NC, _NPAD_A, _H), jnp.float32),
        mesh=mesh,
        scratch_types=[
            pltpu.VMEM((_WIN,), jnp.int32),
            pltpu.VMEM((_WIN,), jnp.int32),
            pltpu.VMEM((_WIN, DE), jnp.float32),
            pltpu.VMEM((_WIN, _H), jnp.float32),
            pltpu.VMEM((_WIN, _H), jnp.float32),
            pltpu.VMEM_SHARED((_NPAD_A, _H), jnp.float32),
            pltpu.SemaphoreType.DMA,
        ],
    )
    return f(exr, m2, src, dst)


def _t16_body(exr_hbm, ef_hbm, didx_hbm, tp_hbm,
              iv, exv, efv, tv, shared, sem):
    cid = lax.axis_index("c")
    sid = lax.axis_index("s")
    wid = sid * _NC + cid
    nwin = (_NWIN - wid - 1) // _NW + 1

    def ztrow(r, c):
        tv[r, pl.ds(0, DE)] = jnp.zeros((_L,), jnp.float32)
        return c

    lax.fori_loop(0, _WIN, ztrow, 0)

    def ztcopy(r, c):
        pltpu.sync_copy(tv, shared.at[pl.ds(sid * _ROWS_T + r * _WIN, _WIN)])
        return c

    lax.fori_loop(0, _ROWS_T // _WIN, ztcopy, 0)
    plsc.subcore_barrier()

    def window(k, carry):
        base = (k * _NW + wid) * _WIN
        pltpu.sync_copy(exr_hbm.at[pl.ds(base, _WIN)], exv)
        pltpu.sync_copy(ef_hbm.at[pl.ds(base, _WIN)], efv)
        pltpu.sync_copy(didx_hbm.at[pl.ds(base, _WIN)], iv)

        def edge(e, c):
            tv[e, pl.ds(0, DE)] = (efv[e, pl.ds(0, DE)] *
                                   exv[e, pl.ds(0, _L)])
            return c

        lax.fori_loop(0, _WIN, edge, 0)
        pltpu.sync_copy(tv, shared.at[iv], add=True)
        return carry

    lax.fori_loop(0, nwin, window, 0)
    plsc.subcore_barrier()
    pltpu.sync_copy(shared.at[pl.ds(sid * _ROWS_T, _ROWS_T)],
                    tp_hbm.at[cid, pl.ds(sid * _ROWS_T, _ROWS_T)])


def _sc_t16(exr, ef, dst):
    mesh = plsc.VectorSubcoreMesh(**_SC_MESH)
    f = pl.kernel(
        _t16_body,
        out_type=jax.ShapeDtypeStruct((_NC, _NPAD, DE), jnp.float32),
        mesh=mesh,
        scratch_types=[
            pltpu.VMEM((_WIN,), jnp.int32),
            pltpu.VMEM((_WIN, DE), jnp.float32),
            pltpu.VMEM((_WIN, DE), jnp.float32),
            pltpu.VMEM((_WIN, DE), jnp.float32),
            pltpu.VMEM_SHARED((_NPAD, DE), jnp.float32),
            pltpu.SemaphoreType.DMA,
        ],
    )
    return f(exr, ef, dst)


def kernel(node_feats, edge_feats,
           W_attn_src, b_attn_src, W_attn_dst, b_attn_dst,
           W_attn_edg, b_attn_edg, W_attn_dot, b_attn_dot,
           W_msg_src, b_msg_src, W_msg_dst, b_msg_dst,
           W_msg_edg, b_msg_edg, W_wgt_n, b_wgt_n,
           edge_index):
    src = edge_index[0]
    dst = edge_index[1]
    x = node_feats.reshape(N, D)

    a_src, a_dst, m_dst, wgt = _node_projections(
        x, W_attn_src, b_attn_src, W_attn_dst, b_attn_dst,
        W_msg_dst, b_msg_dst, W_wgt_n, b_wgt_n)

    u = _sc_gather_add(a_src, a_dst, src, dst)
    scores = _edge_scores(u, edge_feats, W_attn_edg, b_attn_edg,
                          W_attn_dot.reshape(D, 1))

    m16 = jnp.full((_L,), jnp.max(scores), jnp.float32)
    ex, dp = _sc_softmax_denom(scores.reshape(E), m16, dst)
    denom = (dp[0] + dp[1])[:N]

    exr = _ex_replicate(ex.reshape(E, 1))
    m2 = m_dst.reshape(2 * N, _H)
    agg_c = _sc_agg(exr, m2, src, dst)
    t16 = jax.ops.segment_sum(ex[:, None] * edge_feats, dst,
                              num_segments=N)

    out = _final_combine(agg_c[0][:N], agg_c[1][:N], wgt, t16,
                         denom.reshape(N, 1), W_msg_edg, b_msg_edg)
    return out.reshape(N, 1, D)


# speedup vs baseline: 1.2570x; 1.2570x over previous
import jax
import jax.numpy as jnp
from jax import lax
from jax.experimental import pallas as pl
from jax.experimental.pallas import tpu as pltpu
from jax.experimental.pallas import tpu_sc as plsc

N = 10000
E = 160000
D = 256
DE = 16

_NC = 2
_NS = 16
_NW = _NC * _NS
_L = 16

_WIN = 128
_NWIN = E // _WIN
_NPAD = 10240
_ROWS_T = _NPAD // _NS

_NODE_BLK = 1000
_EDGE_BLK = 6400


def _node_proj_body(x_ref, w1, b1, w2, b2, w3, b3, w4, b4,
                    o1, o2, o3, o4):
    x = x_ref[...]
    dn = (((1,), (1,)), ((), ()))
    o1[...] = lax.dot_general(x, w1[...], dn,
                              preferred_element_type=jnp.float32) + b1[...]
    o2[...] = lax.dot_general(x, w2[...], dn,
                              preferred_element_type=jnp.float32) + b2[...]
    o3[...] = lax.dot_general(x, w3[...], dn,
                              preferred_element_type=jnp.float32) + b3[...]
    o4[...] = lax.dot_general(x, w4[...], dn,
                              preferred_element_type=jnp.float32) + b4[...]


def _node_projections(x, W1, b1, W2, b2, W3, b3, W4, b4):
    grid = (N // _NODE_BLK,)
    xspec = pl.BlockSpec((_NODE_BLK, D), lambda i: (i, 0))
    wspec = pl.BlockSpec((D, D), lambda i: (0, 0))
    bspec = pl.BlockSpec((1, D), lambda i: (0, 0))
    ospec = pl.BlockSpec((_NODE_BLK, D), lambda i: (i, 0))
    out_shape = [jax.ShapeDtypeStruct((N, D), jnp.float32)] * 4
    return pl.pallas_call(
        _node_proj_body,
        grid=grid,
        in_specs=[xspec, wspec, bspec, wspec, bspec, wspec, bspec, wspec,
                  bspec],
        out_specs=[ospec] * 4,
        out_shape=out_shape,
    )(x, W1, b1.reshape(1, D), W2, b2.reshape(1, D),
      W3, b3.reshape(1, D), W4, b4.reshape(1, D))


def _edge_score_body(u_ref, ef_ref, w_ref, b_ref, wd_ref, o_ref):
    dn = (((1,), (1,)), ((), ()))
    g = lax.dot_general(ef_ref[...], w_ref[...], dn,
                        preferred_element_type=jnp.float32) + b_ref[...]
    atn = jnp.maximum(u_ref[...] + g, 0.0)
    o_ref[...] = lax.dot_general(atn, wd_ref[...], (((1,), (0,)), ((), ())),
                                 preferred_element_type=jnp.float32)


def _edge_scores(u, ef, W_ae, b_ae, wdot):
    grid = (E // _EDGE_BLK,)
    return pl.pallas_call(
        _edge_score_body,
        grid=grid,
        in_specs=[pl.BlockSpec((_EDGE_BLK, D), lambda i: (i, 0)),
                  pl.BlockSpec((_EDGE_BLK, DE), lambda i: (i, 0)),
                  pl.BlockSpec((D, DE), lambda i: (0, 0)),
                  pl.BlockSpec((1, D), lambda i: (0, 0)),
                  pl.BlockSpec((D, 1), lambda i: (0, 0))],
        out_specs=pl.BlockSpec((_EDGE_BLK, 1), lambda i: (i, 0)),
        out_shape=jax.ShapeDtypeStruct((E, 1), jnp.float32),
    )(u, ef, W_ae, b_ae.reshape(1, D), wdot)


def _msg_body(mg_ref, ef_ref, ex_ref, w_ref, b_ref, o_ref):
    dn = (((1,), (1,)), ((), ()))
    g = lax.dot_general(ef_ref[...], w_ref[...], dn,
                        preferred_element_type=jnp.float32) + b_ref[...]
    o_ref[...] = ex_ref[...] * (mg_ref[...] + g)


def _msg_combine(mg, ef, ex, W_me, b_me):
    blk = 3200
    grid = (E // blk,)
    return pl.pallas_call(
        _msg_body,
        grid=grid,
        in_specs=[pl.BlockSpec((blk, D), lambda i: (i, 0)),
                  pl.BlockSpec((blk, DE), lambda i: (i, 0)),
                  pl.BlockSpec((blk, 1), lambda i: (i, 0)),
                  pl.BlockSpec((D, DE), lambda i: (0, 0)),
                  pl.BlockSpec((1, D), lambda i: (0, 0))],
        out_specs=pl.BlockSpec((blk, D), lambda i: (i, 0)),
        out_shape=jax.ShapeDtypeStruct((E, D), jnp.float32),
    )(mg, ef, ex, W_me, b_me.reshape(1, D))


def _final_body(agg_ref, wgt_ref, den_ref, o_ref):
    r = 1.0 / (den_ref[...] + 1e-12)
    o_ref[...] = wgt_ref[...] + agg_ref[...] * r


def _final_combine(agg, wgt, denom):
    grid = (N // _NODE_BLK,)
    nspec = pl.BlockSpec((_NODE_BLK, D), lambda i: (i, 0))
    return pl.pallas_call(
        _final_body,
        grid=grid,
        in_specs=[nspec, nspec,
                  pl.BlockSpec((_NODE_BLK, 1), lambda i: (i, 0))],
        out_specs=nspec,
        out_shape=jax.ShapeDtypeStruct((N, D), jnp.float32),
    )(agg, wgt, denom)


_SC_MESH = dict(core_axis_name="c", subcore_axis_name="s")


def _gather_add_body(asrc_hbm, adst_hbm, sidx_hbm, didx_hbm, u_hbm,
                     iv1, iv2, ra, rb, uv, sem1, sem2):
    wid = lax.axis_index("s") * _NC + lax.axis_index("c")
    nwin = (_NWIN - wid - 1) // _NW + 1

    def window(k, carry):
        base = (k * _NW + wid) * _WIN
        pltpu.sync_copy(sidx_hbm.at[pl.ds(base, _WIN)], iv1)
        cp1 = pltpu.async_copy(asrc_hbm.at[iv1], ra, sem1)
        pltpu.sync_copy(didx_hbm.at[pl.ds(base, _WIN)], iv2)
        cp2 = pltpu.async_copy(adst_hbm.at[iv2], rb, sem2)
        cp1.wait()
        cp2.wait()

        def edge(e, c):
            for j in range(D // _L):
                uv[e, pl.ds(j * _L, _L)] = (ra[e, pl.ds(j * _L, _L)] +
                                            rb[e, pl.ds(j * _L, _L)])
            return c

        lax.fori_loop(0, _WIN, edge, 0)
        pltpu.sync_copy(uv, u_hbm.at[pl.ds(base, _WIN)])
        return carry

    lax.fori_loop(0, nwin, window, 0)


def _sc_gather_add(a_src, a_dst, src, dst):
    mesh = plsc.VectorSubcoreMesh(**_SC_MESH)
    f = pl.kernel(
        _gather_add_body,
        out_type=jax.ShapeDtypeStruct((E, D), jnp.float32),
        mesh=mesh,
        scratch_types=[
            pltpu.VMEM((_WIN,), jnp.int32),
            pltpu.VMEM((_WIN,), jnp.int32),
            pltpu.VMEM((_WIN, D), jnp.float32),
            pltpu.VMEM((_WIN, D), jnp.float32),
            pltpu.VMEM((_WIN, D), jnp.float32),
            pltpu.SemaphoreType.DMA,
            pltpu.SemaphoreType.DMA,
        ],
    )
    return f(a_src, a_dst, src, dst)


def _softmax_denom_body(s_hbm, m_hbm, didx_hbm, ex_hbm, dp_hbm,
                        mv, sv, ev, iv, zv, shared, sem):
    cid = lax.axis_index("c")
    sid = lax.axis_index("s")
    wid = sid * _NC + cid
    nwin = (_NWIN - wid - 1) // _NW + 1

    def zrow(r, c):
        zv[pl.ds(r * _L, _L)] = jnp.zeros((_L,), jnp.float32)
        return c

    lax.fori_loop(0, _ROWS_T // _L, zrow, 0)
    pltpu.sync_copy(zv, shared.at[pl.ds(sid * _ROWS_T, _ROWS_T)])
    plsc.subcore_barrier()

    pltpu.sync_copy(m_hbm, mv)
    m = mv[...]

    def window(k, carry):
        base = (k * _NW + wid) * _WIN
        pltpu.sync_copy(s_hbm.at[pl.ds(base, _WIN)], sv)
        for q in range(_WIN // _L):
            ev[pl.ds(q * _L, _L)] = jnp.exp(sv[pl.ds(q * _L, _L)] - m)
        pltpu.sync_copy(ev, ex_hbm.at[pl.ds(base, _WIN)])
        pltpu.sync_copy(didx_hbm.at[pl.ds(base, _WIN)], iv)
        pltpu.sync_copy(ev, shared.at[iv], add=True)
        return carry

    lax.fori_loop(0, nwin, window, 0)
    plsc.subcore_barrier()
    pltpu.sync_copy(shared.at[pl.ds(sid * _ROWS_T, _ROWS_T)],
                    dp_hbm.at[cid, pl.ds(sid * _ROWS_T, _ROWS_T)])


def _sc_softmax_denom(scores, m16, dst):
    mesh = plsc.VectorSubcoreMesh(**_SC_MESH)
    f = pl.kernel(
        _softmax_denom_body,
        out_type=[jax.ShapeDtypeStruct((E,), jnp.float32),
                  jax.ShapeDtypeStruct((_NC, _NPAD), jnp.float32)],
        mesh=mesh,
        scratch_types=[
            pltpu.VMEM((_L,), jnp.float32),
            pltpu.VMEM((_WIN,), jnp.float32),
            pltpu.VMEM((_WIN,), jnp.float32),
            pltpu.VMEM((_WIN,), jnp.int32),
            pltpu.VMEM((_ROWS_T,), jnp.float32),
            pltpu.VMEM_SHARED((_NPAD,), jnp.float32),
            pltpu.SemaphoreType.DMA,
        ],
    )
    return f(scores, m16, dst)


def _mgather_body(m_hbm, sidx_hbm, mg_hbm, iv, rows, sem):
    wid = lax.axis_index("s") * _NC + lax.axis_index("c")
    nwin = (_NWIN - wid - 1) // _NW + 1

    def window(k, carry):
        base = (k * _NW + wid) * _WIN
        pltpu.sync_copy(sidx_hbm.at[pl.ds(base, _WIN)], iv)
        pltpu.async_copy(m_hbm.at[iv], rows, sem).wait()
        pltpu.sync_copy(rows, mg_hbm.at[pl.ds(base, _WIN)])
        return carry

    lax.fori_loop(0, nwin, window, 0)


def _sc_mgather(m_dst, src):
    mesh = plsc.VectorSubcoreMesh(**_SC_MESH)
    f = pl.kernel(
        _mgather_body,
        out_type=jax.ShapeDtypeStruct((E, D), jnp.float32),
        mesh=mesh,
        scratch_types=[
            pltpu.VMEM((_WIN,), jnp.int32),
            pltpu.VMEM((_WIN, D), jnp.float32),
            pltpu.SemaphoreType.DMA,
        ],
    )
    return f(m_dst, src)


_H = D // _NC


def _agg_body(msg_hbm, didx_hbm, agg_hbm, iv, rv, zv, shared, sem):
    cid = lax.axis_index("c")
    sid = lax.axis_index("s")
    nwin = (_NWIN - sid - 1) // _NS + 1

    def zrow(r, c):
        for j in range(_H // _L):
            zv[r, pl.ds(j * _L, _L)] = jnp.zeros((_L,), jnp.float32)
        return c

    lax.fori_loop(0, _WIN, zrow, 0)

    def zcopy(r, c):
        pltpu.sync_copy(zv, shared.at[pl.ds(sid * _ROWS_T + r * _WIN, _WIN)])
        return c

    lax.fori_loop(0, _ROWS_T // _WIN, zcopy, 0)
    plsc.subcore_barrier()

    def window(k, carry):
        base = (k * _NS + sid) * _WIN
        pltpu.sync_copy(msg_hbm.at[pl.ds(base, _WIN), pl.ds(cid * _H, _H)],
                        rv)
        pltpu.sync_copy(didx_hbm.at[pl.ds(base, _WIN)], iv)
        pltpu.sync_copy(rv, shared.at[iv], add=True)
        return carry

    lax.fori_loop(0, nwin, window, 0)
    plsc.subcore_barrier()
    pltpu.sync_copy(shared.at[pl.ds(sid * _ROWS_T, _ROWS_T)],
                    agg_hbm.at[pl.ds(sid * _ROWS_T, _ROWS_T),
                               pl.ds(cid * _H, _H)])


def _sc_agg(msg, dst):
    mesh = plsc.VectorSubcoreMesh(**_SC_MESH)
    f = pl.kernel(
        _agg_body,
        out_type=jax.ShapeDtypeStruct((_NPAD, D), jnp.float32),
        mesh=mesh,
        scratch_types=[
            pltpu.VMEM((_WIN,), jnp.int32),
            pltpu.VMEM((_WIN, _H), jnp.float32),
            pltpu.VMEM((_WIN, _H), jnp.float32),
            pltpu.VMEM_SHARED((_NPAD, _H), jnp.float32),
            pltpu.SemaphoreType.DMA,
        ],
    )
    return f(msg, dst)


def kernel(node_feats, edge_feats,
           W_attn_src, b_attn_src, W_attn_dst, b_attn_dst,
           W_attn_edg, b_attn_edg, W_attn_dot, b_attn_dot,
           W_msg_src, b_msg_src, W_msg_dst, b_msg_dst,
           W_msg_edg, b_msg_edg, W_wgt_n, b_wgt_n,
           edge_index):
    src = edge_index[0]
    dst = edge_index[1]
    x = node_feats.reshape(N, D)

    a_src, a_dst, m_dst, wgt = _node_projections(
        x, W_attn_src, b_attn_src, W_attn_dst, b_attn_dst,
        W_msg_dst, b_msg_dst, W_wgt_n, b_wgt_n)

    u = _sc_gather_add(a_src, a_dst, src, dst)
    scores = _edge_scores(u, edge_feats, W_attn_edg, b_attn_edg,
                          W_attn_dot.reshape(D, 1))

    m16 = jnp.full((_L,), jnp.max(scores), jnp.float32)
    ex, dp = _sc_softmax_denom(scores.reshape(E), m16, dst)
    denom = (dp[0] + dp[1])[:N]

    mg = _sc_mgather(m_dst, src)
    msg = _msg_combine(mg, edge_feats, ex.reshape(E, 1),
                       W_msg_edg, b_msg_edg)
    agg = _sc_agg(msg, dst)[:N]

    out = _final_combine(agg, wgt, denom.reshape(N, 1))
    return out.reshape(N, 1, D)
